# baseline (speedup 1.0000x reference)
import jax
import jax.numpy as jnp
from jax import lax
from jax.experimental import pallas as pl
from jax.experimental.pallas import tpu as pltpu
from jax.experimental.pallas import tpu_sc as plsc

VOCAB = 30522
TYPE_VOCAB = 2
HIDDEN = 768
B = 1024
L = 200
N = B * L

NC = 2
NS = 16
NW = NC * NS
LANES = 16

PER_W = N // NW
CHUNK = 32
NCHUNK = PER_W // CHUNK
COLS = HIDDEN // LANES
GRPS = CHUNK // LANES


def _emb_kernel(ids_hbm, ttids_hbm, wtab_hbm, tttab_hbm, out_hbm,
                ids_v, tt_v, tttab_v, g0_v, g1_v, o0_v, o1_v,
                gsem0, gsem1, ssem0, ssem1):
    gbuf = (g0_v, g1_v)
    obuf = (o0_v, o1_v)
    gsem = (gsem0, gsem1)
    ssem = (ssem0, ssem1)

    wid = lax.axis_index("s") * NC + lax.axis_index("c")
    base = wid * PER_W

    pltpu.sync_copy(tttab_hbm, tttab_v)
    pltpu.sync_copy(ids_hbm.at[pl.ds(base, PER_W)], ids_v)
    pltpu.sync_copy(ttids_hbm.at[pl.ds(base, PER_W)], tt_v)

    def fire_gather(g, b):
        pltpu.async_copy(
            wtab_hbm.at[ids_v.at[pl.ds(g * CHUNK, CHUNK)]], gbuf[b], gsem[b])

    def wait_gather(b):
        pltpu.make_async_copy(
            wtab_hbm.at[pl.ds(0, CHUNK)], gbuf[b], gsem[b]).wait()

    def fire_scatter(g, b):
        pltpu.async_copy(
            obuf[b], out_hbm.at[pl.ds(base + g * CHUNK, CHUNK)], ssem[b])

    def wait_scatter(b):
        pltpu.make_async_copy(
            obuf[b], out_hbm.at[pl.ds(base, CHUNK)], ssem[b]).wait()

    def compute(g, b):
        off = g * CHUNK
        gb = gbuf[b]
        ob = obuf[b]

        fb = []
        for g4 in range(GRPS):
            ttf = tt_v[pl.ds(off + g4 * LANES, LANES)].astype(jnp.float32)
            fb += [lax.broadcast(ttf[k], (LANES,)) for k in range(LANES)]

        def col_body(j, c):
            t0 = tttab_v[0, pl.ds(j * LANES, LANES)]
            d = tttab_v[1, pl.ds(j * LANES, LANES)] - t0
            for k in range(CHUNK):
                ob[k, pl.ds(j * LANES, LANES)] = (
                    gb[k, pl.ds(j * LANES, LANES)] + (t0 + fb[k] * d))
            return c

        lax.fori_loop(0, COLS, col_body, 0, unroll=2)

    fire_gather(0, 0)
    fire_gather(1, 1)
    for b in range(2):
        wait_gather(b)
        compute(b, b)
        fire_scatter(b, b)
        fire_gather(b + 2, b)

    @pl.loop(2, NCHUNK - 2, step=2)
    def _(g0):
        for b in range(2):
            g = g0 + b
            wait_gather(b)
            wait_scatter(b)
            compute(g, b)
            fire_scatter(g, b)
            fire_gather(g + 2, b)

    for b in range(2):
        g = NCHUNK - 2 + b
        wait_gather(b)
        wait_scatter(b)
        compute(g, b)
        fire_scatter(g, b)
    for b in range(2):
        wait_scatter(b)


@jax.jit
def kernel(input_ids, token_type_ids, word_embeddings, token_type_embeddings):
    ids = input_ids.reshape(N)
    ttids = token_type_ids.reshape(N)

    mesh = plsc.VectorSubcoreMesh(
        core_axis_name="c", subcore_axis_name="s",
        num_cores=NC, num_subcores=NS)

    run = pl.kernel(
        _emb_kernel,
        out_type=jax.ShapeDtypeStruct((N, HIDDEN), jnp.float32),
        mesh=mesh,
        scratch_types=[
            pltpu.VMEM((PER_W,), jnp.int32),
            pltpu.VMEM((PER_W,), jnp.int32),
            pltpu.VMEM((TYPE_VOCAB, HIDDEN), jnp.float32),
            pltpu.VMEM((CHUNK, HIDDEN), jnp.float32),
            pltpu.VMEM((CHUNK, HIDDEN), jnp.float32),
            pltpu.VMEM((CHUNK, HIDDEN), jnp.float32),
            pltpu.VMEM((CHUNK, HIDDEN), jnp.float32),
            pltpu.SemaphoreType.DMA,
            pltpu.SemaphoreType.DMA,
            pltpu.SemaphoreType.DMA,
            pltpu.SemaphoreType.DMA,
        ],
    )
    out = run(ids, ttids, word_embeddings, token_type_embeddings)
    return out.reshape(B, L, HIDDEN)


# speedup vs baseline: 2.3115x; 2.3115x over previous
import jax
import jax.numpy as jnp
from jax import lax
from jax.experimental import pallas as pl
from jax.experimental.pallas import tpu as pltpu
from jax.experimental.pallas import tpu_sc as plsc

VOCAB = 30522
TYPE_VOCAB = 2
HIDDEN = 768
B = 1024
L = 200
N = B * L

NC = 2
NS = 16
NW = NC * NS
LANES = 16

PER_W = N // NW
CHUNK = 32
NCHUNK = PER_W // CHUNK
COLS = HIDDEN // LANES
GRPS = CHUNK // LANES


def _emb_kernel(ids_hbm, ttids_hbm, wtab_hbm, tttab_hbm, out_hbm,
                ids_v, tt_v, tttab_v, g0_v, g1_v, o0_v, o1_v,
                gsem0, gsem1, ssem0, ssem1):
    gbuf = (g0_v, g1_v)
    obuf = (o0_v, o1_v)
    gsem = (gsem0, gsem1)
    ssem = (ssem0, ssem1)

    wid = lax.axis_index("s") * NC + lax.axis_index("c")
    base = wid * PER_W

    pltpu.sync_copy(tttab_hbm, tttab_v)
    pltpu.sync_copy(ids_hbm.at[pl.ds(base, PER_W)], ids_v)
    pltpu.sync_copy(ttids_hbm.at[pl.ds(base, PER_W)], tt_v)

    def fire_gather(g, b):
        pltpu.async_copy(
            wtab_hbm.at[ids_v.at[pl.ds(g * CHUNK, CHUNK)]], gbuf[b], gsem[b])

    def wait_gather(b):
        pltpu.make_async_copy(
            wtab_hbm.at[pl.ds(0, CHUNK)], gbuf[b], gsem[b]).wait()

    def fire_scatter(g, b):
        pltpu.async_copy(
            obuf[b], out_hbm.at[pl.ds(base + g * CHUNK, CHUNK)], ssem[b])

    def wait_scatter(b):
        pltpu.make_async_copy(
            obuf[b], out_hbm.at[pl.ds(base, CHUNK)], ssem[b]).wait()

    def compute(g, b):
        off = g * CHUNK
        gb = gbuf[b]
        ob = obuf[b]

        def grp_body(g4, c2):
            roff = g4 * LANES
            ttf = tt_v[pl.ds(off + roff, LANES)].astype(jnp.float32)
            fb = [lax.broadcast(ttf[k], (LANES,)) for k in range(LANES)]

            def col_body(j, c):
                t0 = tttab_v[0, pl.ds(j * LANES, LANES)]
                d = tttab_v[1, pl.ds(j * LANES, LANES)] - t0
                for k in range(LANES):
                    ob[roff + k, pl.ds(j * LANES, LANES)] = (
                        gb[roff + k, pl.ds(j * LANES, LANES)]
                        + (t0 + fb[k] * d))
                return c

            lax.fori_loop(0, COLS, col_body, 0, unroll=2)
            return c2

        lax.fori_loop(0, GRPS, grp_body, 0)

    fire_gather(0, 0)
    fire_gather(1, 1)
    for b in range(2):
        wait_gather(b)
        compute(b, b)
        fire_scatter(b, b)
        fire_gather(b + 2, b)

    @pl.loop(2, NCHUNK - 2, step=2)
    def _(g0):
        for b in range(2):
            g = g0 + b
            wait_gather(b)
            wait_scatter(b)
            compute(g, b)
            fire_scatter(g, b)
            fire_gather(g + 2, b)

    for b in range(2):
        g = NCHUNK - 2 + b
        wait_gather(b)
        wait_scatter(b)
        compute(g, b)
        fire_scatter(g, b)
    for b in range(2):
        wait_scatter(b)


@jax.jit
def kernel(input_ids, token_type_ids, word_embeddings, token_type_embeddings):
    ids = input_ids.reshape(N)
    ttids = token_type_ids.reshape(N)

    mesh = plsc.VectorSubcoreMesh(
        core_axis_name="c", subcore_axis_name="s",
        num_cores=NC, num_subcores=NS)

    run = pl.kernel(
        _emb_kernel,
        out_type=jax.ShapeDtypeStruct((N, HIDDEN), jnp.float32),
        mesh=mesh,
        scratch_types=[
            pltpu.VMEM((PER_W,), jnp.int32),
            pltpu.VMEM((PER_W,), jnp.int32),
            pltpu.VMEM((TYPE_VOCAB, HIDDEN), jnp.float32),
            pltpu.VMEM((CHUNK, HIDDEN), jnp.float32),
            pltpu.VMEM((CHUNK, HIDDEN), jnp.float32),
            pltpu.VMEM((CHUNK, HIDDEN), jnp.float32),
            pltpu.VMEM((CHUNK, HIDDEN), jnp.float32),
            pltpu.SemaphoreType.DMA,
            pltpu.SemaphoreType.DMA,
            pltpu.SemaphoreType.DMA,
            pltpu.SemaphoreType.DMA,
        ],
    )
    out = run(ids, ttids, word_embeddings, token_type_embeddings)
    return out.reshape(B, L, HIDDEN)


# speedup vs baseline: 2.4733x; 1.0700x over previous
import jax
import jax.numpy as jnp
from jax import lax
from jax.experimental import pallas as pl
from jax.experimental.pallas import tpu as pltpu
from jax.experimental.pallas import tpu_sc as plsc

VOCAB = 30522
TYPE_VOCAB = 2
HIDDEN = 768
B = 1024
L = 200
N = B * L

NC = 2
NS = 16
NW = NC * NS
LANES = 16

PER_W = N // NW
CHUNK = 32
NCHUNK = PER_W // CHUNK
COLS = HIDDEN // LANES
GRPS = CHUNK // LANES


def _emb_kernel(ids_hbm, ttids_hbm, wtab_hbm, tttab_hbm, out_hbm,
                ids_v, tt_v, tttab_v, g0_v, g1_v, o0_v, o1_v,
                gsem0, gsem1, ssem0, ssem1):
    gbuf = (g0_v, g1_v)
    obuf = (o0_v, o1_v)
    gsem = (gsem0, gsem1)
    ssem = (ssem0, ssem1)

    wid = lax.axis_index("s") * NC + lax.axis_index("c")
    base = wid * PER_W

    pltpu.sync_copy(tttab_hbm, tttab_v)
    pltpu.sync_copy(ids_hbm.at[pl.ds(base, PER_W)], ids_v)
    pltpu.sync_copy(ttids_hbm.at[pl.ds(base, PER_W)], tt_v)

    def fire_gather(g, b):
        pltpu.async_copy(
            wtab_hbm.at[ids_v.at[pl.ds(g * CHUNK, CHUNK)]], gbuf[b], gsem[b])

    def wait_gather(b):
        pltpu.make_async_copy(
            wtab_hbm.at[pl.ds(0, CHUNK)], gbuf[b], gsem[b]).wait()

    def fire_scatter(g, b):
        pltpu.async_copy(
            obuf[b], out_hbm.at[pl.ds(base + g * CHUNK, CHUNK)], ssem[b])

    def wait_scatter(b):
        pltpu.make_async_copy(
            obuf[b], out_hbm.at[pl.ds(base, CHUNK)], ssem[b]).wait()

    def compute(g, b):
        off = g * CHUNK
        gb = gbuf[b]
        ob = obuf[b]

        def grp_body(g4, c2):
            roff = g4 * LANES
            ttf = tt_v[pl.ds(off + roff, LANES)].astype(jnp.float32)
            fb = [lax.broadcast(ttf[k], (LANES,)) for k in range(LANES)]

            def col_body(j, c):
                t0 = tttab_v[0, pl.ds(j * LANES, LANES)]
                d = tttab_v[1, pl.ds(j * LANES, LANES)] - t0
                for k in range(LANES):
                    ob[roff + k, pl.ds(j * LANES, LANES)] = (
                        gb[roff + k, pl.ds(j * LANES, LANES)]
                        + (t0 + fb[k] * d))
                return c

            lax.fori_loop(0, COLS, col_body, 0, unroll=4)
            return c2

        lax.fori_loop(0, GRPS, grp_body, 0)

    fire_gather(0, 0)
    fire_gather(1, 1)
    for b in range(2):
        wait_gather(b)
        compute(b, b)
        fire_scatter(b, b)
        fire_gather(b + 2, b)

    @pl.loop(2, NCHUNK - 2, step=2)
    def _(g0):
        for b in range(2):
            g = g0 + b
            wait_gather(b)
            wait_scatter(b)
            compute(g, b)
            fire_scatter(g, b)
            fire_gather(g + 2, b)

    for b in range(2):
        g = NCHUNK - 2 + b
        wait_gather(b)
        wait_scatter(b)
        compute(g, b)
        fire_scatter(g, b)
    for b in range(2):
        wait_scatter(b)


@jax.jit
def kernel(input_ids, token_type_ids, word_embeddings, token_type_embeddings):
    ids = input_ids.reshape(N)
    ttids = token_type_ids.reshape(N)

    mesh = plsc.VectorSubcoreMesh(
        core_axis_name="c", subcore_axis_name="s",
        num_cores=NC, num_subcores=NS)

    run = pl.kernel(
        _emb_kernel,
        out_type=jax.ShapeDtypeStruct((N, HIDDEN), jnp.float32),
        mesh=mesh,
        scratch_types=[
            pltpu.VMEM((PER_W,), jnp.int32),
            pltpu.VMEM((PER_W,), jnp.int32),
            pltpu.VMEM((TYPE_VOCAB, HIDDEN), jnp.float32),
            pltpu.VMEM((CHUNK, HIDDEN), jnp.float32),
            pltpu.VMEM((CHUNK, HIDDEN), jnp.float32),
            pltpu.VMEM((CHUNK, HIDDEN), jnp.float32),
            pltpu.VMEM((CHUNK, HIDDEN), jnp.float32),
            pltpu.SemaphoreType.DMA,
            pltpu.SemaphoreType.DMA,
            pltpu.SemaphoreType.DMA,
            pltpu.SemaphoreType.DMA,
        ],
    )
    out = run(ids, ttids, word_embeddings, token_type_embeddings)
    return out.reshape(B, L, HIDDEN)
